# Initial kernel scaffold; baseline (speedup 1.0000x reference)
import jax
import jax.numpy as jnp
from jax.experimental import pallas as pl


def kernel(node_features, edge_index, edge_type, graph_ids, W1, b1, W2, b2, W3, b3, Wm, bm):
    raise NotImplementedError("write your pallas kernel here")


# speedup vs baseline: 3.6706x; 3.6706x over previous
import functools

import jax
import jax.numpy as jnp
from jax import lax
from jax.experimental import pallas as pl
from jax.experimental.pallas import tpu as pltpu
from jax.experimental.pallas import tpu_sc as plsc

N = 10000
E = 320000
R = 4
F_IN = 128
H = 256
L_OUT = 56
G = 128

NC = 2
NS = 16
NW = NC * NS

CHUNK = 128
CAP = 16384
NPT = 313
TROWS = 320
JUNK = TROWS - 1
ACC = TROWS * H

BN = 1000
GRID_N = N // BN


def _sc_edge_body(xw_hbm, idxg_hbm, idxd_hbm, nch_hbm, out_hbm,
                  idxg_v, idxd_v, rows_v, acc_v, ncv, sem):
    c = lax.axis_index("c")
    s = lax.axis_index("s")
    w = c * NS + s

    zero16 = jnp.zeros((16,), jnp.float32)

    def zrow(i, carry):
        acc_v[pl.ds(i * 16, 16)] = zero16
        return carry

    lax.fori_loop(0, ACC // 16, zrow, 0)

    pltpu.sync_copy(nch_hbm.at[pl.ds(w * 16, 16)], ncv)
    nchunks = ncv[...][0]

    base = w * CAP

    def chunk_body(j, carry):
        off = base + j * CHUNK
        pltpu.sync_copy(idxg_hbm.at[pl.ds(off, CHUNK)], idxg_v)
        pltpu.sync_copy(idxd_hbm.at[pl.ds(off, CHUNK)], idxd_v)
        pltpu.async_copy(xw_hbm.at[idxg_v], rows_v, sem).wait()

        def gbody(g, carry2):
            dvec = idxd_v[pl.ds(g * 16, 16)]
            for j16 in range(16):
                ld = dvec[j16]
                for k in range(H // 16):
                    plsc.addupdate(acc_v.at[pl.ds(ld + k * 16, 16)],
                                   rows_v[g * 16 + j16, pl.ds(k * 16, 16)])
            return carry2

        lax.fori_loop(0, CHUNK // 16, gbody, 0)
        return carry

    lax.fori_loop(0, nchunks, chunk_body, 0)
    pltpu.sync_copy(acc_v, out_hbm.at[pl.ds(w * ACC, ACC)])


@functools.cache
def _get_sc_edge():
    return pl.kernel(
        _sc_edge_body,
        out_type=jax.ShapeDtypeStruct((NW * ACC,), jnp.float32),
        mesh=plsc.VectorSubcoreMesh(core_axis_name="c", subcore_axis_name="s",
                                    num_cores=NC, num_subcores=NS),
        scratch_types=[
            pltpu.VMEM((CHUNK,), jnp.int32),
            pltpu.VMEM((CHUNK,), jnp.int32),
            pltpu.VMEM((CHUNK, H), jnp.float32),
            pltpu.VMEM((ACC,), jnp.float32),
            pltpu.VMEM((16,), jnp.int32),
            pltpu.SemaphoreType.DMA,
        ],
    )


def _edge_aggregate(xw_flat, idx_g, idx_d, nch):
    out = _get_sc_edge()(xw_flat, idx_g, idx_d, nch)
    return out.reshape(NW, TROWS, H)[:, :NPT, :].reshape(NW * NPT, H)[:N]


def _tc_layer1_body(x_ref, w_ref, xw_ref):
    x = x_ref[...]
    for r in range(R):
        xw_ref[:, r, :] = jnp.dot(x, w_ref[r], preferred_element_type=jnp.float32)


def _tc_layer1(x, w):
    return pl.pallas_call(
        _tc_layer1_body,
        grid=(GRID_N,),
        in_specs=[
            pl.BlockSpec((BN, F_IN), lambda i: (i, 0)),
            pl.BlockSpec((R, F_IN, H), lambda i: (0, 0, 0)),
        ],
        out_specs=pl.BlockSpec((BN, R, H), lambda i: (i, 0, 0)),
        out_shape=jax.ShapeDtypeStruct((N, R, H), jnp.float32),
    )(x, w)


def _tc_layer_body(agg_ref, b_ref, w_ref, xw_ref, h_ref):
    act = jnp.maximum(agg_ref[...] + b_ref[...], 0.0)
    h_ref[...] = act
    for r in range(R):
        xw_ref[:, r, :] = jnp.dot(act, w_ref[r], preferred_element_type=jnp.float32)


def _tc_layer(agg, b, w):
    return pl.pallas_call(
        _tc_layer_body,
        grid=(GRID_N,),
        in_specs=[
            pl.BlockSpec((BN, H), lambda i: (i, 0)),
            pl.BlockSpec((1, H), lambda i: (0, 0)),
            pl.BlockSpec((R, H, H), lambda i: (0, 0, 0)),
        ],
        out_specs=[
            pl.BlockSpec((BN, R, H), lambda i: (i, 0, 0)),
            pl.BlockSpec((BN, H), lambda i: (i, 0)),
        ],
        out_shape=[
            jax.ShapeDtypeStruct((N, R, H), jnp.float32),
            jax.ShapeDtypeStruct((N, H), jnp.float32),
        ],
    )(agg, b, w)


def _tc_final_body(gid_ref, h1_ref, h2_ref, agg3_ref, b3_ref, wm_ref, bm_ref,
                   mu_ref, pooled_ref):
    pid = pl.program_id(0)

    @pl.when(pid == 0)
    def _():
        pooled_ref[...] = jnp.zeros_like(pooled_ref)

    gid = gid_ref[0, 0, :]
    iota = lax.broadcasted_iota(jnp.int32, (G, BN), 0)
    mask = (gid[None, :] == iota).astype(jnp.float32)
    h3 = jnp.maximum(agg3_ref[...] + b3_ref[...], 0.0)
    pooled_ref[:, 0:H] += jnp.dot(mask, h1_ref[...], preferred_element_type=jnp.float32)
    pooled_ref[:, H:2 * H] += jnp.dot(mask, h2_ref[...], preferred_element_type=jnp.float32)
    pooled_ref[:, 2 * H:3 * H] += jnp.dot(mask, h3, preferred_element_type=jnp.float32)

    @pl.when(pid == pl.num_programs(0) - 1)
    def _():
        mu_ref[...] = (jnp.dot(pooled_ref[...], wm_ref[...],
                               preferred_element_type=jnp.float32) + bm_ref[...])


def _tc_final(gid3, h1, h2, agg3, b3, wm, bm):
    return pl.pallas_call(
        _tc_final_body,
        grid=(GRID_N,),
        in_specs=[
            pl.BlockSpec((1, 1, BN), lambda i: (i, 0, 0)),
            pl.BlockSpec((BN, H), lambda i: (i, 0)),
            pl.BlockSpec((BN, H), lambda i: (i, 0)),
            pl.BlockSpec((BN, H), lambda i: (i, 0)),
            pl.BlockSpec((1, H), lambda i: (0, 0)),
            pl.BlockSpec((3 * H, L_OUT), lambda i: (0, 0)),
            pl.BlockSpec((1, L_OUT), lambda i: (0, 0)),
        ],
        out_specs=pl.BlockSpec((G, L_OUT), lambda i: (0, 0)),
        out_shape=jax.ShapeDtypeStruct((G, L_OUT), jnp.float32),
        scratch_shapes=[pltpu.VMEM((G, 3 * H), jnp.float32)],
    )(gid3, h1, h2, agg3, b3, wm, bm)


def kernel(node_features, edge_index, edge_type, graph_ids,
           W1, b1, W2, b2, W3, b3, Wm, bm):
    src = edge_index[0].astype(jnp.int32)
    dst = edge_index[1].astype(jnp.int32)
    et = edge_type.astype(jnp.int32)

    order = jnp.argsort(dst)
    dsts = dst[order]
    gs = (src * R + et)[order]
    tile = dsts // NPT
    ldst = dsts - tile * NPT
    bnd = jnp.searchsorted(dsts, jnp.arange(NW, dtype=jnp.int32) * NPT)
    bnd = bnd.astype(jnp.int32)
    pos = jnp.arange(E, dtype=jnp.int32) - bnd[tile] + tile * CAP
    idx_g = jnp.zeros((NW * CAP,), jnp.int32).at[pos].set(gs)
    idx_d = jnp.full((NW * CAP,), JUNK * H, jnp.int32).at[pos].set(ldst * H)
    cnt = jnp.concatenate([bnd[1:], jnp.array([E], jnp.int32)]) - bnd
    nch = jnp.repeat(-(-cnt // CHUNK), 16)

    gid3 = graph_ids.astype(jnp.int32).reshape(GRID_N, 1, BN)

    xw1 = _tc_layer1(node_features, W1)
    agg1 = _edge_aggregate(xw1.reshape(N * R, H), idx_g, idx_d, nch)
    xw2, h1 = _tc_layer(agg1, b1.reshape(1, H), W2)
    agg2 = _edge_aggregate(xw2.reshape(N * R, H), idx_g, idx_d, nch)
    xw3, h2 = _tc_layer(agg2, b2.reshape(1, H), W3)
    agg3 = _edge_aggregate(xw3.reshape(N * R, H), idx_g, idx_d, nch)
    mu = _tc_final(gid3, h1, h2, agg3, b3.reshape(1, H), Wm,
                   bm.reshape(1, L_OUT))
    return mu


# speedup vs baseline: 5.4885x; 1.4953x over previous
import functools

import jax
import jax.numpy as jnp
from jax import lax
from jax.experimental import pallas as pl
from jax.experimental.pallas import tpu as pltpu
from jax.experimental.pallas import tpu_sc as plsc

N = 10000
E = 320000
R = 4
F_IN = 128
H = 256
L_OUT = 56
G = 128

NC = 2
NS = 16
NW = NC * NS

CHUNK = 128
CAP = 16384
NPT = 313
TROWS = 320
JUNK = TROWS - 1
ACC = TROWS * H

BN = 1000
GRID_N = N // BN


def _sc_edge_body(xw_hbm, idxg_hbm, idxd_hbm, nch_hbm, out_hbm,
                  idxg_v, idxd_v, rows_v, acc_v, ncv, sem):
    c = lax.axis_index("c")
    s = lax.axis_index("s")
    w = c * NS + s

    zero16 = jnp.zeros((16,), jnp.float32)

    def zrow(i, carry):
        acc_v[pl.ds(i * 16, 16)] = zero16
        return carry

    lax.fori_loop(0, ACC // 16, zrow, 0)

    pltpu.sync_copy(nch_hbm.at[pl.ds(w * 16, 16)], ncv)
    nchunks = ncv[...][0]

    base = w * CAP

    def chunk_body(j, carry):
        off = base + j * CHUNK
        pltpu.sync_copy(idxg_hbm.at[pl.ds(off, CHUNK)], idxg_v)
        pltpu.sync_copy(idxd_hbm.at[pl.ds(off, CHUNK)], idxd_v)
        pltpu.async_copy(xw_hbm.at[idxg_v], rows_v, sem).wait()

        def gbody(g, carry2):
            dvec = idxd_v[pl.ds(g * 16, 16)]
            for j16 in range(16):
                ld = dvec[j16]
                for k in range(H // 16):
                    plsc.addupdate(acc_v.at[pl.ds(ld + k * 16, 16)],
                                   rows_v[g * 16 + j16, pl.ds(k * 16, 16)])
            return carry2

        lax.fori_loop(0, CHUNK // 16, gbody, 0)
        return carry

    lax.fori_loop(0, nchunks, chunk_body, 0)
    pltpu.sync_copy(acc_v, out_hbm.at[pl.ds(w * ACC, ACC)])


@functools.cache
def _get_sc_edge():
    return pl.kernel(
        _sc_edge_body,
        out_type=jax.ShapeDtypeStruct((NW * ACC,), jnp.float32),
        mesh=plsc.VectorSubcoreMesh(core_axis_name="c", subcore_axis_name="s",
                                    num_cores=NC, num_subcores=NS),
        scratch_types=[
            pltpu.VMEM((CHUNK,), jnp.int32),
            pltpu.VMEM((CHUNK,), jnp.int32),
            pltpu.VMEM((CHUNK, H), jnp.float32),
            pltpu.VMEM((ACC,), jnp.float32),
            pltpu.VMEM((16,), jnp.int32),
            pltpu.SemaphoreType.DMA,
        ],
    )


def _edge_aggregate(xw_flat, idx_g, idx_d, nch):
    out = _get_sc_edge()(xw_flat, idx_g, idx_d, nch)
    return out.reshape(NW, TROWS, H)[:, :NPT, :].reshape(NW * NPT, H)[:N]


def _tc_layer1_body(x_ref, w_ref, xw_ref):
    x = x_ref[...]
    for r in range(R):
        xw_ref[:, r, :] = jnp.dot(x, w_ref[r], preferred_element_type=jnp.float32)


def _tc_layer1(x, w):
    return pl.pallas_call(
        _tc_layer1_body,
        grid=(GRID_N,),
        in_specs=[
            pl.BlockSpec((BN, F_IN), lambda i: (i, 0)),
            pl.BlockSpec((R, F_IN, H), lambda i: (0, 0, 0)),
        ],
        out_specs=pl.BlockSpec((BN, R, H), lambda i: (i, 0, 0)),
        out_shape=jax.ShapeDtypeStruct((N, R, H), jnp.float32),
    )(x, w)


def _tc_layer_body(agg_ref, b_ref, w_ref, xw_ref, h_ref):
    act = jnp.maximum(agg_ref[...] + b_ref[...], 0.0)
    h_ref[...] = act
    for r in range(R):
        xw_ref[:, r, :] = jnp.dot(act, w_ref[r], preferred_element_type=jnp.float32)


def _tc_layer(agg, b, w):
    return pl.pallas_call(
        _tc_layer_body,
        grid=(GRID_N,),
        in_specs=[
            pl.BlockSpec((BN, H), lambda i: (i, 0)),
            pl.BlockSpec((1, H), lambda i: (0, 0)),
            pl.BlockSpec((R, H, H), lambda i: (0, 0, 0)),
        ],
        out_specs=[
            pl.BlockSpec((BN, R, H), lambda i: (i, 0, 0)),
            pl.BlockSpec((BN, H), lambda i: (i, 0)),
        ],
        out_shape=[
            jax.ShapeDtypeStruct((N, R, H), jnp.float32),
            jax.ShapeDtypeStruct((N, H), jnp.float32),
        ],
    )(agg, b, w)


def _tc_final_body(gid_ref, h1_ref, h2_ref, agg3_ref, b3_ref, wm_ref, bm_ref,
                   mu_ref, pooled_ref):
    pid = pl.program_id(0)

    @pl.when(pid == 0)
    def _():
        pooled_ref[...] = jnp.zeros_like(pooled_ref)

    gid = gid_ref[0, 0, :]
    iota = lax.broadcasted_iota(jnp.int32, (G, BN), 0)
    mask = (gid[None, :] == iota).astype(jnp.float32)
    h3 = jnp.maximum(agg3_ref[...] + b3_ref[...], 0.0)
    pooled_ref[:, 0:H] += jnp.dot(mask, h1_ref[...], preferred_element_type=jnp.float32)
    pooled_ref[:, H:2 * H] += jnp.dot(mask, h2_ref[...], preferred_element_type=jnp.float32)
    pooled_ref[:, 2 * H:3 * H] += jnp.dot(mask, h3, preferred_element_type=jnp.float32)

    @pl.when(pid == pl.num_programs(0) - 1)
    def _():
        mu_ref[...] = (jnp.dot(pooled_ref[...], wm_ref[...],
                               preferred_element_type=jnp.float32) + bm_ref[...])


def _tc_final(gid3, h1, h2, agg3, b3, wm, bm):
    return pl.pallas_call(
        _tc_final_body,
        grid=(GRID_N,),
        in_specs=[
            pl.BlockSpec((1, 1, BN), lambda i: (i, 0, 0)),
            pl.BlockSpec((BN, H), lambda i: (i, 0)),
            pl.BlockSpec((BN, H), lambda i: (i, 0)),
            pl.BlockSpec((BN, H), lambda i: (i, 0)),
            pl.BlockSpec((1, H), lambda i: (0, 0)),
            pl.BlockSpec((3 * H, L_OUT), lambda i: (0, 0)),
            pl.BlockSpec((1, L_OUT), lambda i: (0, 0)),
        ],
        out_specs=pl.BlockSpec((G, L_OUT), lambda i: (0, 0)),
        out_shape=jax.ShapeDtypeStruct((G, L_OUT), jnp.float32),
        scratch_shapes=[pltpu.VMEM((G, 3 * H), jnp.float32)],
    )(gid3, h1, h2, agg3, b3, wm, bm)


def kernel(node_features, edge_index, edge_type, graph_ids,
           W1, b1, W2, b2, W3, b3, Wm, bm):
    src = edge_index[0].astype(jnp.int32)
    dst = edge_index[1].astype(jnp.int32)
    et = edge_type.astype(jnp.int32)

    order = jnp.argsort(dst)
    dsts = dst[order]
    gs = (src * R + et)[order]
    bnd = jnp.searchsorted(dsts, jnp.arange(NW, dtype=jnp.int32) * NPT)
    bnd = bnd.astype(jnp.int32)
    bnd2 = jnp.concatenate([bnd[1:], jnp.array([E], jnp.int32)])
    iv = jnp.arange(NW * CAP, dtype=jnp.int32)
    wv = iv // CAP
    jv = bnd[wv] + (iv & (CAP - 1))
    valid = jv < bnd2[wv]
    jc = jnp.minimum(jv, E - 1)
    idx_g = jnp.where(valid, gs[jc], 0)
    idx_d = jnp.where(valid, (dsts[jc] - wv * NPT) * H, JUNK * H)
    cnt = bnd2 - bnd
    nch = jnp.repeat(-(-cnt // CHUNK), 16)

    gid3 = graph_ids.astype(jnp.int32).reshape(GRID_N, 1, BN)

    xw1 = _tc_layer1(node_features, W1)
    agg1 = _edge_aggregate(xw1.reshape(N * R, H), idx_g, idx_d, nch)
    xw2, h1 = _tc_layer(agg1, b1.reshape(1, H), W2)
    agg2 = _edge_aggregate(xw2.reshape(N * R, H), idx_g, idx_d, nch)
    xw3, h2 = _tc_layer(agg2, b2.reshape(1, H), W3)
    agg3 = _edge_aggregate(xw3.reshape(N * R, H), idx_g, idx_d, nch)
    mu = _tc_final(gid3, h1, h2, agg3, b3.reshape(1, H), Wm,
                   bm.reshape(1, L_OUT))
    return mu


# speedup vs baseline: 6.3814x; 1.1627x over previous
import functools

import jax
import jax.numpy as jnp
from jax import lax
from jax.experimental import pallas as pl
from jax.experimental.pallas import tpu as pltpu
from jax.experimental.pallas import tpu_sc as plsc

N = 10000
E = 320000
R = 4
F_IN = 128
H = 256
L_OUT = 56
G = 128

NC = 2
NS = 16
NW = NC * NS

CH = 64
CAP = 16384
NPT = 313
TROWS = 320
JUNK = TROWS - 1
ACC = TROWS * H

BN = 1000
GRID_N = N // BN


def _sc_edge_body(xw_hbm, idx2_hbm, nch_hbm, out_hbm,
                  i0, i1, r0, r1, acc_v, ncv, si0, si1, sr0, sr1):
    c = lax.axis_index("c")
    s = lax.axis_index("s")
    w = c * NS + s

    zero16 = jnp.zeros((16,), jnp.float32)

    def zrow(i, carry):
        for k in range(16):
            acc_v[pl.ds(i * 256 + k * 16, 16)] = zero16
        return carry

    lax.fori_loop(0, ACC // 256, zrow, 0)

    pltpu.sync_copy(nch_hbm.at[pl.ds(w * 16, 16)], ncv)
    nc = ncv[...][0]

    base2 = w * 2 * CAP

    def issue_idx(j, ib, sem):
        pltpu.async_copy(idx2_hbm.at[pl.ds(base2 + j * 2 * CH, 2 * CH)], ib, sem)

    def wait_idx(ib, sem):
        pltpu.make_async_copy(idx2_hbm.at[pl.ds(0, 2 * CH)], ib, sem).wait()

    def start_gather(ib, rb, sem):
        pltpu.async_copy(xw_hbm.at[ib.at[pl.ds(0, CH)]], rb, sem)

    def wait_gather(ib, rb, sem):
        pltpu.make_async_copy(xw_hbm.at[ib.at[pl.ds(0, CH)]], rb, sem).wait()

    def accumulate(ib, rb):
        def gbody(g, carry):
            dvec = ib[pl.ds(CH + g * 16, 16)]
            for j16 in range(16):
                ld = dvec[j16]
                for k in range(H // 16):
                    plsc.addupdate(acc_v.at[pl.ds(ld + k * 16, 16)],
                                   rb[g * 16 + j16, pl.ds(k * 16, 16)])
            return carry

        lax.fori_loop(0, CH // 16, gbody, 0)

    bufs = ((i0, r0, si0, sr0), (i1, r1, si1, sr1))

    @pl.when(nc > 0)
    def _():
        issue_idx(0, i0, si0)

    @pl.when(nc > 1)
    def _():
        issue_idx(1, i1, si1)

    @pl.when(nc > 0)
    def _():
        wait_idx(i0, si0)
        start_gather(i0, r0, sr0)

    def pair(jj, carry):
        for b in (0, 1):
            j = 2 * jj + b
            ib, rb, si, sr = bufs[b]
            ob, orb, osi, osr = bufs[1 - b]

            @pl.when(j + 1 < nc)
            def _():
                wait_idx(ob, osi)
                start_gather(ob, orb, osr)

            wait_gather(ib, rb, sr)
            accumulate(ib, rb)

            @pl.when(j + 2 < nc)
            def _():
                issue_idx(j + 2, ib, si)
        return carry

    lax.fori_loop(0, (nc + 1) // 2, pair, 0)
    pltpu.sync_copy(acc_v, out_hbm.at[pl.ds(w * ACC, ACC)])


@functools.cache
def _get_sc_edge():
    return pl.kernel(
        _sc_edge_body,
        out_type=jax.ShapeDtypeStruct((NW * ACC,), jnp.float32),
        mesh=plsc.VectorSubcoreMesh(core_axis_name="c", subcore_axis_name="s",
                                    num_cores=NC, num_subcores=NS),
        scratch_types=[
            pltpu.VMEM((2 * CH,), jnp.int32),
            pltpu.VMEM((2 * CH,), jnp.int32),
            pltpu.VMEM((CH, H), jnp.float32),
            pltpu.VMEM((CH, H), jnp.float32),
            pltpu.VMEM((ACC,), jnp.float32),
            pltpu.VMEM((16,), jnp.int32),
            pltpu.SemaphoreType.DMA,
            pltpu.SemaphoreType.DMA,
            pltpu.SemaphoreType.DMA,
            pltpu.SemaphoreType.DMA,
        ],
    )


def _edge_aggregate(xw_flat, idx2, nch):
    out = _get_sc_edge()(xw_flat, idx2, nch)
    return out.reshape(NW, TROWS, H)[:, :NPT, :].reshape(NW * NPT, H)[:N]


def _tc_layer1_body(x_ref, w_ref, xw_ref):
    x = x_ref[...]
    for r in range(R):
        xw_ref[r] = jnp.dot(x, w_ref[r], preferred_element_type=jnp.float32)


def _tc_layer1(x, w):
    return pl.pallas_call(
        _tc_layer1_body,
        grid=(GRID_N,),
        in_specs=[
            pl.BlockSpec((BN, F_IN), lambda i: (i, 0)),
            pl.BlockSpec((R, F_IN, H), lambda i: (0, 0, 0)),
        ],
        out_specs=pl.BlockSpec((R, BN, H), lambda i: (0, i, 0)),
        out_shape=jax.ShapeDtypeStruct((R, N, H), jnp.float32),
    )(x, w)


def _tc_layer_body(agg_ref, b_ref, w_ref, xw_ref, h_ref):
    act = jnp.maximum(agg_ref[...] + b_ref[...], 0.0)
    h_ref[...] = act
    for r in range(R):
        xw_ref[r] = jnp.dot(act, w_ref[r], preferred_element_type=jnp.float32)


def _tc_layer(agg, b, w):
    return pl.pallas_call(
        _tc_layer_body,
        grid=(GRID_N,),
        in_specs=[
            pl.BlockSpec((BN, H), lambda i: (i, 0)),
            pl.BlockSpec((1, H), lambda i: (0, 0)),
            pl.BlockSpec((R, H, H), lambda i: (0, 0, 0)),
        ],
        out_specs=[
            pl.BlockSpec((R, BN, H), lambda i: (0, i, 0)),
            pl.BlockSpec((BN, H), lambda i: (i, 0)),
        ],
        out_shape=[
            jax.ShapeDtypeStruct((R, N, H), jnp.float32),
            jax.ShapeDtypeStruct((N, H), jnp.float32),
        ],
    )(agg, b, w)


def _tc_final_body(gid_ref, h1_ref, h2_ref, agg3_ref, b3_ref, wm_ref, bm_ref,
                   mu_ref, pooled_ref):
    pid = pl.program_id(0)

    @pl.when(pid == 0)
    def _():
        pooled_ref[...] = jnp.zeros_like(pooled_ref)

    gid = gid_ref[0, 0, :]
    iota = lax.broadcasted_iota(jnp.int32, (G, BN), 0)
    mask = (gid[None, :] == iota).astype(jnp.float32)
    h3 = jnp.maximum(agg3_ref[...] + b3_ref[...], 0.0)
    pooled_ref[:, 0:H] += jnp.dot(mask, h1_ref[...], preferred_element_type=jnp.float32)
    pooled_ref[:, H:2 * H] += jnp.dot(mask, h2_ref[...], preferred_element_type=jnp.float32)
    pooled_ref[:, 2 * H:3 * H] += jnp.dot(mask, h3, preferred_element_type=jnp.float32)

    @pl.when(pid == pl.num_programs(0) - 1)
    def _():
        mu_ref[...] = (jnp.dot(pooled_ref[...], wm_ref[...],
                               preferred_element_type=jnp.float32) + bm_ref[...])


def _tc_final(gid3, h1, h2, agg3, b3, wm, bm):
    return pl.pallas_call(
        _tc_final_body,
        grid=(GRID_N,),
        in_specs=[
            pl.BlockSpec((1, 1, BN), lambda i: (i, 0, 0)),
            pl.BlockSpec((BN, H), lambda i: (i, 0)),
            pl.BlockSpec((BN, H), lambda i: (i, 0)),
            pl.BlockSpec((BN, H), lambda i: (i, 0)),
            pl.BlockSpec((1, H), lambda i: (0, 0)),
            pl.BlockSpec((3 * H, L_OUT), lambda i: (0, 0)),
            pl.BlockSpec((1, L_OUT), lambda i: (0, 0)),
        ],
        out_specs=pl.BlockSpec((G, L_OUT), lambda i: (0, 0)),
        out_shape=jax.ShapeDtypeStruct((G, L_OUT), jnp.float32),
        scratch_shapes=[pltpu.VMEM((G, 3 * H), jnp.float32)],
    )(gid3, h1, h2, agg3, b3, wm, bm)


def kernel(node_features, edge_index, edge_type, graph_ids,
           W1, b1, W2, b2, W3, b3, Wm, bm):
    src = edge_index[0].astype(jnp.int32)
    dst = edge_index[1].astype(jnp.int32)
    et = edge_type.astype(jnp.int32)

    order = jnp.argsort(dst)
    dsts = dst[order]
    gs = (et * N + src)[order]
    bnd = jnp.searchsorted(dsts, jnp.arange(NW, dtype=jnp.int32) * NPT)
    bnd = bnd.astype(jnp.int32)
    bnd2 = jnp.concatenate([bnd[1:], jnp.array([E], jnp.int32)])
    iv = jnp.arange(NW * CAP, dtype=jnp.int32)
    wv = iv // CAP
    jv = bnd[wv] + (iv & (CAP - 1))
    valid = jv < bnd2[wv]
    jc = jnp.minimum(jv, E - 1)
    g_lay = jnp.where(valid, gs[jc], 0)
    d_lay = jnp.where(valid, (dsts[jc] - wv * NPT) * H, JUNK * H)
    idx2 = jnp.stack([g_lay.reshape(-1, CH), d_lay.reshape(-1, CH)],
                     axis=1).reshape(-1)
    cnt = bnd2 - bnd
    nch = jnp.repeat(2 * (-(-cnt // (2 * CH))), 16)

    gid3 = graph_ids.astype(jnp.int32).reshape(GRID_N, 1, BN)

    xw1 = _tc_layer1(node_features, W1)
    agg1 = _edge_aggregate(xw1.reshape(R * N, H), idx2, nch)
    xw2, h1 = _tc_layer(agg1, b1.reshape(1, H), W2)
    agg2 = _edge_aggregate(xw2.reshape(R * N, H), idx2, nch)
    xw3, h2 = _tc_layer(agg2, b2.reshape(1, H), W3)
    agg3 = _edge_aggregate(xw3.reshape(R * N, H), idx2, nch)
    mu = _tc_final(gid3, h1, h2, agg3, b3.reshape(1, H), Wm,
                   bm.reshape(1, L_OUT))
    return mu


# speedup vs baseline: 6.7132x; 1.0520x over previous
import functools

import jax
import jax.numpy as jnp
from jax import lax
from jax.experimental import pallas as pl
from jax.experimental.pallas import tpu as pltpu
from jax.experimental.pallas import tpu_sc as plsc

N = 10000
E = 320000
R = 4
F_IN = 128
H = 256
L_OUT = 56
G = 128

NC = 2
NS = 16
NW = NC * NS

CH = 64
CAP = 16384
NPT = 313
TROWS = 320
JUNK = TROWS - 1
ACC = TROWS * H

BN = 1000
GRID_N = N // BN


def _sc_edge_body(xw_hbm, idx2_hbm, nch_hbm, out_hbm,
                  i0, i1, r0, r1, acc_v, ncv, si0, si1, sr0, sr1):
    c = lax.axis_index("c")
    s = lax.axis_index("s")
    w = c * NS + s

    zero16 = jnp.zeros((16,), jnp.float32)

    def zrow(i, carry):
        for k in range(16):
            acc_v[pl.ds(i * 256 + k * 16, 16)] = zero16
        return carry

    lax.fori_loop(0, ACC // 256, zrow, 0)

    pltpu.sync_copy(nch_hbm.at[pl.ds(w * 16, 16)], ncv)
    nc = ncv[...][0]

    base2 = w * 2 * CAP

    def issue_idx(j, ib, sem):
        pltpu.async_copy(idx2_hbm.at[pl.ds(base2 + j * 2 * CH, 2 * CH)], ib, sem)

    def wait_idx(ib, sem):
        pltpu.make_async_copy(idx2_hbm.at[pl.ds(0, 2 * CH)], ib, sem).wait()

    def start_gather(ib, rb, sem):
        pltpu.async_copy(xw_hbm.at[ib.at[pl.ds(0, CH)]], rb, sem)

    def wait_gather(ib, rb, sem):
        pltpu.make_async_copy(xw_hbm.at[ib.at[pl.ds(0, CH)]], rb, sem).wait()

    def accumulate(ib, rb):
        def gbody(g, carry):
            dvec = ib[pl.ds(CH + g * 16, 16)]
            for j16 in range(16):
                ld = dvec[j16]
                for k in range(H // 16):
                    plsc.addupdate(acc_v.at[pl.ds(ld + k * 16, 16)],
                                   rb[g * 16 + j16, pl.ds(k * 16, 16)])
            return carry

        lax.fori_loop(0, CH // 16, gbody, 0)

    bufs = ((i0, r0, si0, sr0), (i1, r1, si1, sr1))

    @pl.when(nc > 0)
    def _():
        issue_idx(0, i0, si0)

    @pl.when(nc > 1)
    def _():
        issue_idx(1, i1, si1)

    @pl.when(nc > 0)
    def _():
        wait_idx(i0, si0)
        start_gather(i0, r0, sr0)

    def pair(jj, carry):
        for b in (0, 1):
            j = 2 * jj + b
            ib, rb, si, sr = bufs[b]
            ob, orb, osi, osr = bufs[1 - b]

            @pl.when(j + 1 < nc)
            def _():
                wait_idx(ob, osi)
                start_gather(ob, orb, osr)

            wait_gather(ib, rb, sr)
            accumulate(ib, rb)

            @pl.when(j + 2 < nc)
            def _():
                issue_idx(j + 2, ib, si)
        return carry

    lax.fori_loop(0, (nc + 1) // 2, pair, 0)
    pltpu.sync_copy(acc_v, out_hbm.at[pl.ds(w * ACC, ACC)])


@functools.cache
def _get_sc_edge():
    return pl.kernel(
        _sc_edge_body,
        out_type=jax.ShapeDtypeStruct((NW * ACC,), jnp.float32),
        mesh=plsc.VectorSubcoreMesh(core_axis_name="c", subcore_axis_name="s",
                                    num_cores=NC, num_subcores=NS),
        scratch_types=[
            pltpu.VMEM((2 * CH,), jnp.int32),
            pltpu.VMEM((2 * CH,), jnp.int32),
            pltpu.VMEM((CH, H), jnp.float32),
            pltpu.VMEM((CH, H), jnp.float32),
            pltpu.VMEM((ACC,), jnp.float32),
            pltpu.VMEM((16,), jnp.int32),
            pltpu.SemaphoreType.DMA,
            pltpu.SemaphoreType.DMA,
            pltpu.SemaphoreType.DMA,
            pltpu.SemaphoreType.DMA,
        ],
    )


def _edge_aggregate(xw_flat, idx2, nch):
    out = _get_sc_edge()(xw_flat, idx2, nch)
    return out.reshape(NW, TROWS, H)[:, :NPT, :].reshape(NW * NPT, H)[:N]


def _tc_layer1_body(x_ref, w_ref, xw_ref):
    x = x_ref[...]
    for r in range(R):
        xw_ref[r] = jnp.dot(x, w_ref[r], preferred_element_type=jnp.float32)


def _tc_layer1(x, w):
    return pl.pallas_call(
        _tc_layer1_body,
        grid=(GRID_N,),
        in_specs=[
            pl.BlockSpec((BN, F_IN), lambda i: (i, 0)),
            pl.BlockSpec((R, F_IN, H), lambda i: (0, 0, 0)),
        ],
        out_specs=pl.BlockSpec((R, BN, H), lambda i: (0, i, 0)),
        out_shape=jax.ShapeDtypeStruct((R, N, H), jnp.float32),
    )(x, w)


def _tc_layer_body(agg_ref, b_ref, w_ref, xw_ref, h_ref):
    act = jnp.maximum(agg_ref[...] + b_ref[...], 0.0)
    h_ref[...] = act
    for r in range(R):
        xw_ref[r] = jnp.dot(act, w_ref[r], preferred_element_type=jnp.float32)


def _tc_layer(agg, b, w):
    return pl.pallas_call(
        _tc_layer_body,
        grid=(GRID_N,),
        in_specs=[
            pl.BlockSpec((BN, H), lambda i: (i, 0)),
            pl.BlockSpec((1, H), lambda i: (0, 0)),
            pl.BlockSpec((R, H, H), lambda i: (0, 0, 0)),
        ],
        out_specs=[
            pl.BlockSpec((R, BN, H), lambda i: (0, i, 0)),
            pl.BlockSpec((BN, H), lambda i: (i, 0)),
        ],
        out_shape=[
            jax.ShapeDtypeStruct((R, N, H), jnp.float32),
            jax.ShapeDtypeStruct((N, H), jnp.float32),
        ],
    )(agg, b, w)


def _tc_final_body(gid_ref, h1_ref, h2_ref, agg3_ref, b3_ref, wm_ref, bm_ref,
                   mu_ref, pooled_ref):
    pid = pl.program_id(0)

    @pl.when(pid == 0)
    def _():
        pooled_ref[...] = jnp.zeros_like(pooled_ref)

    gid = gid_ref[0, 0, :]
    iota = lax.broadcasted_iota(jnp.int32, (G, BN), 0)
    mask = (gid[None, :] == iota).astype(jnp.float32)
    h3 = jnp.maximum(agg3_ref[...] + b3_ref[...], 0.0)
    pooled_ref[:, 0:H] += jnp.dot(mask, h1_ref[...], preferred_element_type=jnp.float32)
    pooled_ref[:, H:2 * H] += jnp.dot(mask, h2_ref[...], preferred_element_type=jnp.float32)
    pooled_ref[:, 2 * H:3 * H] += jnp.dot(mask, h3, preferred_element_type=jnp.float32)

    @pl.when(pid == pl.num_programs(0) - 1)
    def _():
        mu_ref[...] = (jnp.dot(pooled_ref[...], wm_ref[...],
                               preferred_element_type=jnp.float32) + bm_ref[...])


def _tc_final(gid3, h1, h2, agg3, b3, wm, bm):
    return pl.pallas_call(
        _tc_final_body,
        grid=(GRID_N,),
        in_specs=[
            pl.BlockSpec((1, 1, BN), lambda i: (i, 0, 0)),
            pl.BlockSpec((BN, H), lambda i: (i, 0)),
            pl.BlockSpec((BN, H), lambda i: (i, 0)),
            pl.BlockSpec((BN, H), lambda i: (i, 0)),
            pl.BlockSpec((1, H), lambda i: (0, 0)),
            pl.BlockSpec((3 * H, L_OUT), lambda i: (0, 0)),
            pl.BlockSpec((1, L_OUT), lambda i: (0, 0)),
        ],
        out_specs=pl.BlockSpec((G, L_OUT), lambda i: (0, 0)),
        out_shape=jax.ShapeDtypeStruct((G, L_OUT), jnp.float32),
        scratch_shapes=[pltpu.VMEM((G, 3 * H), jnp.float32)],
    )(gid3, h1, h2, agg3, b3, wm, bm)


def kernel(node_features, edge_index, edge_type, graph_ids,
           W1, b1, W2, b2, W3, b3, Wm, bm):
    src = edge_index[0].astype(jnp.int32)
    dst = edge_index[1].astype(jnp.int32)
    et = edge_type.astype(jnp.int32)

    dsts, gs = lax.sort([dst, et * N + src], dimension=0, num_keys=1,
                        is_stable=False)
    bnd = jnp.sum(dsts[None, :] < (jnp.arange(NW, dtype=jnp.int32) * NPT)[:, None],
                  axis=1, dtype=jnp.int32)
    bnd2 = jnp.concatenate([bnd[1:], jnp.array([E], jnp.int32)])
    iv = jnp.arange(NW * CAP, dtype=jnp.int32)
    wv = iv // CAP
    jv = bnd[wv] + (iv & (CAP - 1))
    valid = jv < bnd2[wv]
    jc = jnp.minimum(jv, E - 1)
    g_lay = jnp.where(valid, gs[jc], 0)
    d_lay = jnp.where(valid, (dsts[jc] - wv * NPT) * H, JUNK * H)
    idx2 = jnp.stack([g_lay.reshape(-1, CH), d_lay.reshape(-1, CH)],
                     axis=1).reshape(-1)
    cnt = bnd2 - bnd
    nch = jnp.repeat(2 * (-(-cnt // (2 * CH))), 16)

    gid3 = graph_ids.astype(jnp.int32).reshape(GRID_N, 1, BN)

    xw1 = _tc_layer1(node_features, W1)
    agg1 = _edge_aggregate(xw1.reshape(R * N, H), idx2, nch)
    xw2, h1 = _tc_layer(agg1, b1.reshape(1, H), W2)
    agg2 = _edge_aggregate(xw2.reshape(R * N, H), idx2, nch)
    xw3, h2 = _tc_layer(agg2, b2.reshape(1, H), W3)
    agg3 = _edge_aggregate(xw3.reshape(R * N, H), idx2, nch)
    mu = _tc_final(gid3, h1, h2, agg3, b3.reshape(1, H), Wm,
                   bm.reshape(1, L_OUT))
    return mu
